# baseline (speedup 1.0000x reference)
import functools

import jax
import jax.numpy as jnp
from jax import lax
from jax.experimental import pallas as pl
from jax.experimental.pallas import tpu as pltpu
from jax.experimental.pallas import tpu_sc as plsc

B = 16384
ED = 10
NC, NS = 2, 16
NW = NC * NS
BPW = B // NW
CHUNK = 128
NCH = BPW // CHUNK


def _gather5(uid2, tid2, tca2, sid2, sca2, tabs):
    mesh = plsc.VectorSubcoreMesh(core_axis_name="c", subcore_axis_name="s")
    out_t = [jax.ShapeDtypeStruct((ED, B), jnp.float32)] * 5
    scratch = ([pltpu.VMEM((NCH, CHUNK), jnp.int32) for _ in range(5)]
               + [pltpu.VMEM((ED, BPW), jnp.float32) for _ in range(5)]
               + [pltpu.SemaphoreType.DMA])

    @functools.partial(pl.kernel, out_type=out_t, mesh=mesh,
                       scratch_types=scratch,
                       compiler_params=pltpu.CompilerParams(
                           use_tc_tiling_on_sc=False))
    def k(uid_h, tid_h, tca_h, sid_h, sca_h,
          t0, t1, t2, t3, t4,
          o0, o1, o2, o3, o4,
          i0, i1, i2, i3, i4, r0, r1, r2, r3, r4, sem):
        wid = lax.axis_index("s") * NC + lax.axis_index("c")
        idx_hs = (uid_h, tid_h, tca_h, sid_h, sca_h)
        idx_vs = (i0, i1, i2, i3, i4)
        row_vs = (r0, r1, r2, r3, r4)
        tab_hs = (t0, t1, t2, t3, t4)
        outs = (o0, o1, o2, o3, o4)
        for t in range(5):
            pltpu.sync_copy(idx_hs[t].at[pl.ds(wid * NCH, NCH)], idx_vs[t])
        cps = []
        for t in range(5):
            for c in range(ED):
                for j in range(NCH):
                    cps.append(pltpu.async_copy(
                        tab_hs[t].at[c].at[idx_vs[t].at[j]],
                        row_vs[t].at[c, pl.ds(j * CHUNK, CHUNK)], sem))
        for cp in cps:
            cp.wait()
        for t in range(5):
            pltpu.sync_copy(row_vs[t], outs[t].at[:, pl.ds(wid * BPW, BPW)])

    return k(uid2, tid2, tca2, sid2, sca2, *tabs)


def _mlp_body(eu, ti, tc, si, sc,
              su, wsm, wsh, tu, wtm, wth, hm, hh,
              ws1, h1, wt1, ws2, h2, wt2, ws3, h3, wt3,
              sw, sb, tw, tb, rs, rt):
    d = lambda w, x: lax.dot_general(w[...], x, (((1,), (0,)), ((), ())),
                                     preferred_element_type=jnp.float32)
    eu_, ti_, tc_, si_, sc_ = eu[...], ti[...], tc[...], si[...], sc[...]
    a_s = d(su, eu_) + d(wsm, si_) + d(wsh, sc_) + d(hm, ti_) + d(hh, tc_)
    a_t = d(tu, eu_) + d(wtm, ti_) + d(wth, tc_) + d(hm, si_) + d(hh, sc_)
    xs = jnp.maximum(a_s, 0.0)
    xt = jnp.maximum(a_t, 0.0)
    for (w, h, wt) in ((ws1, h1, wt1), (ws2, h2, wt2), (ws3, h3, wt3)):
        ns = jnp.maximum(d(w, xs) + d(h, xt), 0.0)
        nt = jnp.maximum(d(wt, xt) + d(h, xs), 0.0)
        xs, xt = ns, nt
    ls = d(sw, xs) + sb[...]
    lt = d(tw, xt) + tb[...]
    rs[...] = 1.0 / (1.0 + jnp.exp(-ls))
    rt[...] = 1.0 / (1.0 + jnp.exp(-lt))


def _mlp(eu, ti, tc, si, sc, mats, sw, sb, tw, tb, interpret=False):
    BB = 2048
    grid = (B // BB,)
    dspec = pl.BlockSpec((ED, BB), lambda i: (0, i))
    wspec = lambda a: pl.BlockSpec(a.shape, lambda i: (0, 0))
    in_specs = ([dspec] * 5 + [wspec(m) for m in mats]
                + [wspec(sw), wspec(sb), wspec(tw), wspec(tb)])
    out_specs = [pl.BlockSpec((1, BB), lambda i: (0, i))] * 2
    out_shape = [jax.ShapeDtypeStruct((1, B), jnp.float32)] * 2
    return pl.pallas_call(
        _mlp_body, grid=grid, in_specs=in_specs, out_specs=out_specs,
        out_shape=out_shape, interpret=interpret,
    )(eu, ti, tc, si, sc, *mats, sw, sb, tw, tb)


def kernel(userid, t_can_id, t_can_cate, s_can_id, s_can_cate,
           user_emb, t_itemid_emb, t_itemcate_emb, s_itemid_emb, s_itemcate_emb,
           ws0, h0, wt0, ws1, h1, wt1, ws2, h2, wt2, ws3, h3, wt3,
           s_pred_w, s_pred_b, t_pred_w, t_pred_b):
    uid2 = userid.reshape(B // CHUNK, CHUNK)
    tid2 = t_can_id.reshape(B // CHUNK, CHUNK)
    tca2 = t_can_cate.reshape(B // CHUNK, CHUNK)
    sid2 = s_can_id.reshape(B // CHUNK, CHUNK)
    sca2 = s_can_cate.reshape(B // CHUNK, CHUNK)
    tabs = [t.T for t in (user_emb, t_itemid_emb, t_itemcate_emb,
                          s_itemid_emb, s_itemcate_emb)]
    eu, ti, tc, si, sc = _gather5(uid2, tid2, tca2, sid2, sca2, tabs)
    mats = (ws0[:, :ED] + h0[:, :ED],
            ws0[:, ED:2 * ED], ws0[:, 2 * ED:],
            wt0[:, :ED] + h0[:, :ED],
            wt0[:, ED:2 * ED], wt0[:, 2 * ED:],
            h0[:, ED:2 * ED], h0[:, 2 * ED:],
            ws1, h1, wt1, ws2, h2, wt2, ws3, h3, wt3)
    rs, rt = _mlp(eu, ti, tc, si, sc, mats,
                  s_pred_w, s_pred_b.reshape(1, 1),
                  t_pred_w, t_pred_b.reshape(1, 1))
    return rs.reshape(B), rt.reshape(B)


# speedup vs baseline: 10.8743x; 10.8743x over previous
import functools

import jax
import jax.numpy as jnp
from jax import lax
from jax.experimental import pallas as pl
from jax.experimental.pallas import tpu as pltpu
from jax.experimental.pallas import tpu_sc as plsc

B = 16384
ED = 10
NC, NS = 2, 16
NW = NC * NS
BPW = B // NW
CHUNK = 128
NCH = BPW // CHUNK


CW = 8192


def _vp(v0):
    return -(-v0 // 128) * 128


def _detile(tts):
    mesh = plsc.VectorSubcoreMesh(core_axis_name="c", subcore_axis_name="s")
    v0s = [t.shape[1] - 1 for t in tts]
    out_t = [jax.ShapeDtypeStruct((ED * _vp(v0),), jnp.float32) for v0 in v0s]

    @functools.partial(pl.kernel, out_type=out_t, mesh=mesh,
                       scratch_types=[pltpu.VMEM((ED, CW), jnp.float32)],
                       compiler_params=pltpu.CompilerParams(
                           use_tc_tiling_on_sc=True))
    def k(t0, t1, t2, t3, t4, o0, o1, o2, o3, o4, buf):
        wid = lax.axis_index("s") * NC + lax.axis_index("c")
        for t, (tab, out, v0) in enumerate(zip((t0, t1, t2, t3, t4),
                                               (o0, o1, o2, o3, o4), v0s)):
            v128 = (v0 // 128) * 128
            vp = _vp(v0)
            nfull = v128 // CW
            tail = v128 - nfull * CW

            def do_chunk(m, width, tab=tab, out=out, vp=vp):
                pltpu.sync_copy(tab.at[:, pl.ds(m * CW, width)],
                                buf.at[:, pl.ds(0, width)])
                for c in range(ED):
                    pltpu.sync_copy(buf.at[c, pl.ds(0, width)],
                                    out.at[pl.ds(c * vp + m * CW, width)])

            if nfull >= NW:
                for g in range((nfull + NW - 1) // NW):
                    m = jnp.minimum(wid + NW * g, nfull - 1)
                    do_chunk(m, CW)
            else:
                @pl.when(wid < nfull)
                def _():
                    do_chunk(wid, CW)
            if tail:
                @pl.when(wid == (t % NW))
                def _():
                    do_chunk(nfull, tail)

    return k(*tts)


def _tails(tts, flats):
    v0s = [t.shape[1] - 1 for t in tts]
    nb = [t.shape[1] // 128 for t in tts]
    in_specs = ([pl.BlockSpec(memory_space=pl.ANY) for _ in flats]
                + [pl.BlockSpec((ED, 128),
                                functools.partial(lambda n, i: (0, n), n))
                   for n in nb])
    out_specs = [pl.BlockSpec((128,),
                              functools.partial(
                                  lambda vp, n, i: (i * (vp // 128) + n,),
                                  _vp(v0), n))
                 for v0, n in zip(v0s, nb)]
    out_shape = [jax.ShapeDtypeStruct(c.shape, c.dtype) for c in flats]

    def body(*refs):
        tabs = refs[5:10]
        outs = refs[10:]
        c = pl.program_id(0)
        for t in range(5):
            outs[t][...] = jnp.squeeze(tabs[t][pl.ds(c, 1), :], axis=0)

    return pl.pallas_call(
        body, grid=(ED,), in_specs=in_specs, out_specs=out_specs,
        out_shape=out_shape,
        input_output_aliases={i: i for i in range(5)},
    )(*flats, *tts)


def _gather5(uid2, tid2, tca2, sid2, sca2, tabs):
    mesh = plsc.VectorSubcoreMesh(core_axis_name="c", subcore_axis_name="s")
    out_t = [jax.ShapeDtypeStruct((ED, B), jnp.float32)] * 5
    scratch = ([pltpu.VMEM((NCH, CHUNK), jnp.int32) for _ in range(5)]
               + [pltpu.VMEM((ED, BPW), jnp.float32) for _ in range(5)]
               + [pltpu.SemaphoreType.DMA])

    @functools.partial(pl.kernel, out_type=out_t, mesh=mesh,
                       scratch_types=scratch,
                       compiler_params=pltpu.CompilerParams(
                           use_tc_tiling_on_sc=False))
    def k(uid_h, tid_h, tca_h, sid_h, sca_h,
          t0, t1, t2, t3, t4,
          o0, o1, o2, o3, o4,
          i0, i1, i2, i3, i4, r0, r1, r2, r3, r4, sem):
        wid = lax.axis_index("s") * NC + lax.axis_index("c")
        idx_hs = (uid_h, tid_h, tca_h, sid_h, sca_h)
        idx_vs = (i0, i1, i2, i3, i4)
        row_vs = (r0, r1, r2, r3, r4)
        tab_hs = (t0, t1, t2, t3, t4)
        outs = (o0, o1, o2, o3, o4)
        for t in range(5):
            pltpu.sync_copy(idx_hs[t].at[pl.ds(wid * NCH, NCH)], idx_vs[t])
        cps = []
        for t in range(5):
            v0 = tab_hs[t].shape[0] // ED
            for c in range(ED):
                for j in range(NCH):
                    cps.append(pltpu.async_copy(
                        tab_hs[t].at[pl.ds(c * v0, v0)].at[idx_vs[t].at[j]],
                        row_vs[t].at[c, pl.ds(j * CHUNK, CHUNK)], sem))
        for cp in cps:
            cp.wait()
        for t in range(5):
            pltpu.sync_copy(row_vs[t], outs[t].at[:, pl.ds(wid * BPW, BPW)])

    return k(uid2, tid2, tca2, sid2, sca2, *tabs)


def _mlp_body(eu, ti, tc, si, sc,
              su, wsm, wsh, tu, wtm, wth, hm, hh,
              ws1, h1, wt1, ws2, h2, wt2, ws3, h3, wt3,
              sw, sb, tw, tb, rs, rt):
    d = lambda w, x: lax.dot_general(w[...], x, (((1,), (0,)), ((), ())),
                                     preferred_element_type=jnp.float32)
    eu_, ti_, tc_, si_, sc_ = eu[...], ti[...], tc[...], si[...], sc[...]
    a_s = d(su, eu_) + d(wsm, si_) + d(wsh, sc_) + d(hm, ti_) + d(hh, tc_)
    a_t = d(tu, eu_) + d(wtm, ti_) + d(wth, tc_) + d(hm, si_) + d(hh, sc_)
    xs = jnp.maximum(a_s, 0.0)
    xt = jnp.maximum(a_t, 0.0)
    for (w, h, wt) in ((ws1, h1, wt1), (ws2, h2, wt2), (ws3, h3, wt3)):
        ns = jnp.maximum(d(w, xs) + d(h, xt), 0.0)
        nt = jnp.maximum(d(wt, xt) + d(h, xs), 0.0)
        xs, xt = ns, nt
    ls = d(sw, xs) + sb[...]
    lt = d(tw, xt) + tb[...]
    rs[...] = 1.0 / (1.0 + jnp.exp(-ls))
    rt[...] = 1.0 / (1.0 + jnp.exp(-lt))


def _mlp(eu, ti, tc, si, sc, mats, sw, sb, tw, tb, interpret=False):
    BB = 2048
    grid = (B // BB,)
    dspec = pl.BlockSpec((ED, BB), lambda i: (0, i))
    wspec = lambda a: pl.BlockSpec(a.shape, lambda i: (0, 0))
    in_specs = ([dspec] * 5 + [wspec(m) for m in mats]
                + [wspec(sw), wspec(sb), wspec(tw), wspec(tb)])
    out_specs = [pl.BlockSpec((1, BB), lambda i: (0, i))] * 2
    out_shape = [jax.ShapeDtypeStruct((1, B), jnp.float32)] * 2
    return pl.pallas_call(
        _mlp_body, grid=grid, in_specs=in_specs, out_specs=out_specs,
        out_shape=out_shape, interpret=interpret,
    )(eu, ti, tc, si, sc, *mats, sw, sb, tw, tb)


def kernel(userid, t_can_id, t_can_cate, s_can_id, s_can_cate,
           user_emb, t_itemid_emb, t_itemcate_emb, s_itemid_emb, s_itemcate_emb,
           ws0, h0, wt0, ws1, h1, wt1, ws2, h2, wt2, ws3, h3, wt3,
           s_pred_w, s_pred_b, t_pred_w, t_pred_b):
    uid2 = userid.reshape(B // CHUNK, CHUNK)
    tid2 = t_can_id.reshape(B // CHUNK, CHUNK)
    tca2 = t_can_cate.reshape(B // CHUNK, CHUNK)
    sid2 = s_can_id.reshape(B // CHUNK, CHUNK)
    sca2 = s_can_cate.reshape(B // CHUNK, CHUNK)
    tts = [t.T for t in (user_emb, t_itemid_emb, t_itemcate_emb,
                         s_itemid_emb, s_itemcate_emb)]
    tabs = _tails(tts, _detile(tts))
    eu, ti, tc, si, sc = _gather5(uid2, tid2, tca2, sid2, sca2, tabs)
    mats = (ws0[:, :ED] + h0[:, :ED],
            ws0[:, ED:2 * ED], ws0[:, 2 * ED:],
            wt0[:, :ED] + h0[:, :ED],
            wt0[:, ED:2 * ED], wt0[:, 2 * ED:],
            h0[:, ED:2 * ED], h0[:, 2 * ED:],
            ws1, h1, wt1, ws2, h2, wt2, ws3, h3, wt3)
    rs, rt = _mlp(eu, ti, tc, si, sc, mats,
                  s_pred_w, s_pred_b.reshape(1, 1),
                  t_pred_w, t_pred_b.reshape(1, 1))
    return rs.reshape(B), rt.reshape(B)


# speedup vs baseline: 10.9436x; 1.0064x over previous
import functools

import jax
import jax.numpy as jnp
from jax import lax
from jax.experimental import pallas as pl
from jax.experimental.pallas import tpu as pltpu
from jax.experimental.pallas import tpu_sc as plsc

B = 16384
ED = 10
NC, NS = 2, 16
NW = NC * NS
BPW = B // NW
CHUNK = 128
NCH = BPW // CHUNK


CW = 8192


def _vp(v0):
    return -(-v0 // 128) * 128


def _detile(tts):
    mesh = plsc.VectorSubcoreMesh(core_axis_name="c", subcore_axis_name="s")
    v0s = [t.shape[1] - 1 for t in tts]
    out_t = [jax.ShapeDtypeStruct((ED * _vp(v0),), jnp.float32) for v0 in v0s]

    @functools.partial(pl.kernel, out_type=out_t, mesh=mesh,
                       scratch_types=[pltpu.VMEM((ED, CW), jnp.float32)],
                       compiler_params=pltpu.CompilerParams(
                           use_tc_tiling_on_sc=True))
    def k(t0, t1, t2, t3, t4, o0, o1, o2, o3, o4, buf):
        wid = lax.axis_index("s") * NC + lax.axis_index("c")
        for t, (tab, out, v0) in enumerate(zip((t0, t1, t2, t3, t4),
                                               (o0, o1, o2, o3, o4), v0s)):
            v128 = (v0 // 128) * 128
            vp = _vp(v0)
            nfull = v128 // CW
            tail = v128 - nfull * CW

            def do_chunk(m, width, tab=tab, out=out, vp=vp):
                pltpu.sync_copy(tab.at[:, pl.ds(m * CW, width)],
                                buf.at[:, pl.ds(0, width)])
                for c in range(ED):
                    pltpu.sync_copy(buf.at[c, pl.ds(0, width)],
                                    out.at[pl.ds(c * vp + m * CW, width)])

            if nfull >= NW:
                for g in range((nfull + NW - 1) // NW):
                    m = jnp.minimum(wid + NW * g, nfull - 1)
                    do_chunk(m, CW)
            else:
                @pl.when(wid < nfull)
                def _():
                    do_chunk(wid, CW)
            if tail:
                @pl.when(wid == (t % NW))
                def _():
                    do_chunk(nfull, tail)

    return k(*tts)


def _tails(tts, flats):
    v0s = [t.shape[1] - 1 for t in tts]
    nb = [t.shape[1] // 128 for t in tts]
    in_specs = ([pl.BlockSpec(memory_space=pl.ANY) for _ in flats]
                + [pl.BlockSpec((ED, 128),
                                functools.partial(lambda n, i: (0, n), n))
                   for n in nb])
    out_specs = [pl.BlockSpec((128,),
                              functools.partial(
                                  lambda vp, n, i: (i * (vp // 128) + n,),
                                  _vp(v0), n))
                 for v0, n in zip(v0s, nb)]
    out_shape = [jax.ShapeDtypeStruct(c.shape, c.dtype) for c in flats]

    def body(*refs):
        tabs = refs[5:10]
        outs = refs[10:]
        c = pl.program_id(0)
        for t in range(5):
            outs[t][...] = jnp.squeeze(tabs[t][pl.ds(c, 1), :], axis=0)

    return pl.pallas_call(
        body, grid=(ED,), in_specs=in_specs, out_specs=out_specs,
        out_shape=out_shape,
        input_output_aliases={i: i for i in range(5)},
    )(*flats, *tts)


def _gather5(uid2, tid2, tca2, sid2, sca2, tabs):
    mesh = plsc.VectorSubcoreMesh(core_axis_name="c", subcore_axis_name="s")
    out_t = [jax.ShapeDtypeStruct((ED, B), jnp.float32)] * 5
    scratch = ([pltpu.VMEM((BPW,), jnp.int32) for _ in range(5)]
               + [pltpu.VMEM((ED, BPW), jnp.float32) for _ in range(5)]
               + [pltpu.SemaphoreType.DMA])

    @functools.partial(pl.kernel, out_type=out_t, mesh=mesh,
                       scratch_types=scratch,
                       compiler_params=pltpu.CompilerParams(
                           use_tc_tiling_on_sc=False))
    def k(uid_h, tid_h, tca_h, sid_h, sca_h,
          t0, t1, t2, t3, t4,
          o0, o1, o2, o3, o4,
          i0, i1, i2, i3, i4, r0, r1, r2, r3, r4, sem):
        wid = lax.axis_index("s") * NC + lax.axis_index("c")
        idx_hs = (uid_h, tid_h, tca_h, sid_h, sca_h)
        idx_vs = (i0, i1, i2, i3, i4)
        row_vs = (r0, r1, r2, r3, r4)
        tab_hs = (t0, t1, t2, t3, t4)
        outs = (o0, o1, o2, o3, o4)
        for t in range(5):
            pltpu.sync_copy(idx_hs[t].at[pl.ds(wid * BPW, BPW)], idx_vs[t])
        cps = []
        for t in range(5):
            v0 = tab_hs[t].shape[0] // ED
            for c in range(ED):
                cps.append(pltpu.async_copy(
                    tab_hs[t].at[pl.ds(c * v0, v0)].at[idx_vs[t]],
                    row_vs[t].at[c], sem))
        for cp in cps:
            cp.wait()
        for t in range(5):
            pltpu.sync_copy(row_vs[t], outs[t].at[:, pl.ds(wid * BPW, BPW)])

    return k(uid2, tid2, tca2, sid2, sca2, *tabs)


def _mlp_body(eu, ti, tc, si, sc,
              su, wsm, wsh, tu, wtm, wth, hm, hh,
              ws1, h1, wt1, ws2, h2, wt2, ws3, h3, wt3,
              sw, sb, tw, tb, rs, rt):
    d = lambda w, x: lax.dot_general(w[...], x, (((1,), (0,)), ((), ())),
                                     preferred_element_type=jnp.float32)
    eu_, ti_, tc_, si_, sc_ = eu[...], ti[...], tc[...], si[...], sc[...]
    a_s = d(su, eu_) + d(wsm, si_) + d(wsh, sc_) + d(hm, ti_) + d(hh, tc_)
    a_t = d(tu, eu_) + d(wtm, ti_) + d(wth, tc_) + d(hm, si_) + d(hh, sc_)
    xs = jnp.maximum(a_s, 0.0)
    xt = jnp.maximum(a_t, 0.0)
    for (w, h, wt) in ((ws1, h1, wt1), (ws2, h2, wt2), (ws3, h3, wt3)):
        ns = jnp.maximum(d(w, xs) + d(h, xt), 0.0)
        nt = jnp.maximum(d(wt, xt) + d(h, xs), 0.0)
        xs, xt = ns, nt
    ls = d(sw, xs) + sb[...]
    lt = d(tw, xt) + tb[...]
    rs[...] = 1.0 / (1.0 + jnp.exp(-ls))
    rt[...] = 1.0 / (1.0 + jnp.exp(-lt))


def _mlp(eu, ti, tc, si, sc, mats, sw, sb, tw, tb, interpret=False):
    BB = 2048
    grid = (B // BB,)
    dspec = pl.BlockSpec((ED, BB), lambda i: (0, i))
    wspec = lambda a: pl.BlockSpec(a.shape, lambda i: (0, 0))
    in_specs = ([dspec] * 5 + [wspec(m) for m in mats]
                + [wspec(sw), wspec(sb), wspec(tw), wspec(tb)])
    out_specs = [pl.BlockSpec((1, BB), lambda i: (0, i))] * 2
    out_shape = [jax.ShapeDtypeStruct((1, B), jnp.float32)] * 2
    return pl.pallas_call(
        _mlp_body, grid=grid, in_specs=in_specs, out_specs=out_specs,
        out_shape=out_shape, interpret=interpret,
    )(eu, ti, tc, si, sc, *mats, sw, sb, tw, tb)


def kernel(userid, t_can_id, t_can_cate, s_can_id, s_can_cate,
           user_emb, t_itemid_emb, t_itemcate_emb, s_itemid_emb, s_itemcate_emb,
           ws0, h0, wt0, ws1, h1, wt1, ws2, h2, wt2, ws3, h3, wt3,
           s_pred_w, s_pred_b, t_pred_w, t_pred_b):
    tts = [t.T for t in (user_emb, t_itemid_emb, t_itemcate_emb,
                         s_itemid_emb, s_itemcate_emb)]
    tabs = _tails(tts, _detile(tts))
    eu, ti, tc, si, sc = _gather5(userid, t_can_id, t_can_cate,
                                  s_can_id, s_can_cate, tabs)
    mats = (ws0[:, :ED] + h0[:, :ED],
            ws0[:, ED:2 * ED], ws0[:, 2 * ED:],
            wt0[:, :ED] + h0[:, :ED],
            wt0[:, ED:2 * ED], wt0[:, 2 * ED:],
            h0[:, ED:2 * ED], h0[:, 2 * ED:],
            ws1, h1, wt1, ws2, h2, wt2, ws3, h3, wt3)
    rs, rt = _mlp(eu, ti, tc, si, sc, mats,
                  s_pred_w, s_pred_b.reshape(1, 1),
                  t_pred_w, t_pred_b.reshape(1, 1))
    return rs.reshape(B), rt.reshape(B)
